# Initial kernel scaffold; baseline (speedup 1.0000x reference)
import jax
import jax.numpy as jnp
from jax.experimental import pallas as pl


def kernel(embed_table, w1, b1, w2, b2, wc, bc, masks, code_mask, leaves_list, ancestors_list, input_ids):
    raise NotImplementedError("write your pallas kernel here")


# speedup vs baseline: 1.1723x; 1.1723x over previous
import functools

import jax
import jax.numpy as jnp
from jax.experimental import pallas as pl
from jax.experimental.pallas import tpu as pltpu

H = 768
A = 8
N_CODES = 10000
OUT = 512
VERY_NEG = -1e30

TM = 1000
TN = 400
TB = 64


def _project_kernel(e_ref, w_ref, b_ref, ql_ref, qae_ref):
    eb = e_ref[...].astype(jnp.bfloat16)
    p = jnp.dot(eb, w_ref[...], preferred_element_type=jnp.float32)
    p = p + b_ref[...]
    ql_ref[...] = p[:, :H].astype(jnp.bfloat16)
    qae_ref[...] = jnp.concatenate(
        [p[:, H:].astype(jnp.bfloat16), eb], axis=-1)


def _project(embed_table, w_cat, b_cat):
    n = embed_table.shape[0]
    grid = (n // TM,)
    return pl.pallas_call(
        _project_kernel,
        grid=grid,
        in_specs=[
            pl.BlockSpec((TM, H), lambda i: (i, 0)),
            pl.BlockSpec((H, 2 * H), lambda i: (0, 0)),
            pl.BlockSpec((1, 2 * H), lambda i: (0, 0)),
        ],
        out_specs=[
            pl.BlockSpec((TM, H), lambda i: (i, 0)),
            pl.BlockSpec((TM, 2 * H), lambda i: (i, 0)),
        ],
        out_shape=[
            jax.ShapeDtypeStruct((n, H), jnp.bfloat16),
            jax.ShapeDtypeStruct((n, 2 * H), jnp.bfloat16),
        ],
        compiler_params=pltpu.CompilerParams(
            dimension_semantics=("parallel",),
            vmem_limit_bytes=100 * 1024 * 1024,
        ),
    )(embed_table, w_cat, b_cat)


def _attn_kernel(gl_ref, gae_ref, m_ref, w2b_ref, wc_ref, out_ref):
    gl = gl_ref[...]
    gae = gae_ref[...]
    h = jnp.concatenate(
        [
            jnp.maximum(
                gl[:, a * H:(a + 1) * H] + gae[:, a * 2 * H:a * 2 * H + H],
                jnp.bfloat16(0.0),
            )
            for a in range(A)
        ],
        axis=-1,
    )
    logits = jnp.dot(h, w2b_ref[...], preferred_element_type=jnp.float32)
    logits = jnp.where(m_ref[...] > 0.0, logits, VERY_NEG)
    mx = jnp.max(logits, axis=-1, keepdims=True)
    e = jnp.exp(logits - mx)
    attn = e / jnp.sum(e, axis=-1, keepdims=True)
    dag = functools.reduce(
        lambda x, y: x + y,
        [
            attn[:, a:a + 1]
            * gae[:, a * 2 * H + H:(a + 1) * 2 * H].astype(jnp.float32)
            for a in range(A)
        ],
    )
    out_ref[...] = jnp.dot(
        dag.astype(jnp.bfloat16), wc_ref[...],
        preferred_element_type=jnp.float32)


def _attention(gl, gae, masks, w2_blk, wc_b):
    grid = (N_CODES // TN,)
    return pl.pallas_call(
        _attn_kernel,
        grid=grid,
        in_specs=[
            pl.BlockSpec((TN, A * H), lambda i: (i, 0)),
            pl.BlockSpec((TN, A * 2 * H), lambda i: (i, 0)),
            pl.BlockSpec((TN, A), lambda i: (i, 0)),
            pl.BlockSpec((A * H, A), lambda i: (0, 0)),
            pl.BlockSpec((H, OUT), lambda i: (0, 0)),
        ],
        out_specs=pl.BlockSpec((TN, OUT), lambda i: (i, 0)),
        out_shape=jax.ShapeDtypeStruct((N_CODES, OUT), jnp.float32),
        compiler_params=pltpu.CompilerParams(
            dimension_semantics=("parallel",),
            vmem_limit_bytes=100 * 1024 * 1024,
        ),
    )(gl, gae, masks, w2_blk, wc_b)


P4 = OUT // 128


def _pool_kernel(idx_ref, wgt_ref, cm_ref, c4_ref, bc_ref, out_ref):
    def row(r, _):
        acc = jnp.zeros((P4, 128), jnp.float32)
        cnt = jnp.float32(0.0)
        for m in range(48):
            i4 = pl.multiple_of(idx_ref[r, m], P4)
            acc = acc + wgt_ref[r, m] * c4_ref[pl.ds(i4, P4), :]
            cnt = cnt + cm_ref[r, m]
        scale = 1.0 / jnp.maximum(jnp.full((P4, 128), cnt), 1.0)
        out_ref[pl.ds(r * P4, P4), :] = acc * scale + bc_ref[...]
        return ()

    jax.lax.fori_loop(0, TB, row, ())


def _pool(idx4, wgt, cmf, c4, bc4):
    bv = idx4.shape[0]
    grid = (bv // TB,)
    return pl.pallas_call(
        _pool_kernel,
        grid=grid,
        in_specs=[
            pl.BlockSpec((TB, 48), lambda i: (i, 0),
                         memory_space=pltpu.SMEM),
            pl.BlockSpec((TB, 48), lambda i: (i, 0),
                         memory_space=pltpu.SMEM),
            pl.BlockSpec((TB, 48), lambda i: (i, 0),
                         memory_space=pltpu.SMEM),
            pl.BlockSpec((N_CODES * P4, 128), lambda i: (0, 0)),
            pl.BlockSpec((P4, 128), lambda i: (0, 0)),
        ],
        out_specs=pl.BlockSpec((TB * P4, 128), lambda i: (i, 0)),
        out_shape=jax.ShapeDtypeStruct((bv * P4, 128), jnp.float32),
        compiler_params=pltpu.CompilerParams(
            dimension_semantics=("parallel",),
            vmem_limit_bytes=100 * 1024 * 1024,
        ),
    )(idx4, wgt, cmf, c4, bc4)


def kernel(embed_table, w1, b1, w2, b2, wc, bc, masks, code_mask,
           leaves_list, ancestors_list, input_ids):
    del b2
    f32 = jnp.float32
    w_cat = jnp.concatenate([w1[:H, :], w1[H:, :]], axis=1).astype(jnp.bfloat16)
    b_cat = jnp.concatenate([jnp.zeros((H,), f32), b1]).reshape(1, 2 * H)
    ql, qae = _project(embed_table, w_cat, b_cat)

    gl = ql[leaves_list.astype(jnp.int32)].reshape(N_CODES, A * H)
    gae = qae[ancestors_list.astype(jnp.int32)].reshape(N_CODES, A * 2 * H)

    w2b = jnp.zeros((A, H, A), f32).at[jnp.arange(A), :, jnp.arange(A)].set(
        w2[:, 0][None, :]).reshape(A * H, A).astype(jnp.bfloat16)
    c = _attention(gl, gae, masks, w2b, wc.astype(jnp.bfloat16))

    ids = input_ids.reshape(-1, 48).astype(jnp.int32)
    idx4 = jnp.where(ids == 0, 0, (ids - 1) * P4)
    cmf = code_mask.reshape(-1, 48).astype(f32)
    wgt = cmf * (ids != 0).astype(f32)
    out = _pool(idx4, wgt, cmf, c.reshape(N_CODES * P4, 128),
                bc.reshape(P4, 128))
    B, V, _ = input_ids.shape
    return out.reshape(B, V, OUT)


# speedup vs baseline: 1.3240x; 1.1294x over previous
import functools

import jax
import jax.numpy as jnp
from jax.experimental import pallas as pl
from jax.experimental.pallas import tpu as pltpu

H = 768
A = 8
AE = H + 512
N_CODES = 10000
OUT = 512
VERY_NEG = -1e30

TM = 1000
TN = 200
NCHUNK = 5
TB = 64


def _project_kernel(e_ref, w_ref, b_ref, ql_ref, qae_ref):
    eb = e_ref[...].astype(jnp.bfloat16)
    p = jnp.dot(eb, w_ref[...], preferred_element_type=jnp.float32)
    p = p + b_ref[...]
    ql_ref[...] = p[:, :H].astype(jnp.bfloat16)
    qae_ref[...] = p[:, H:].astype(jnp.bfloat16)


def _project(embed_table, w_cat, b_cat):
    n = embed_table.shape[0]
    return pl.pallas_call(
        _project_kernel,
        grid=(n // TM,),
        in_specs=[
            pl.BlockSpec((TM, H), lambda i: (i, 0)),
            pl.BlockSpec((H, H + AE), lambda i: (0, 0)),
            pl.BlockSpec((1, H + AE), lambda i: (0, 0)),
        ],
        out_specs=[
            pl.BlockSpec((TM, H), lambda i: (i, 0)),
            pl.BlockSpec((TM, AE), lambda i: (i, 0)),
        ],
        out_shape=[
            jax.ShapeDtypeStruct((n, H), jnp.bfloat16),
            jax.ShapeDtypeStruct((n, AE), jnp.bfloat16),
        ],
        compiler_params=pltpu.CompilerParams(
            dimension_semantics=("parallel",),
            vmem_limit_bytes=100 * 1024 * 1024,
        ),
    )(embed_table, w_cat, b_cat)


def _attn_kernel(gl_ref, gae_ref, m_ref, w2b_ref, out_ref):
    gl = gl_ref[...]
    gae = gae_ref[...]
    h = jnp.concatenate(
        [
            jnp.maximum(
                gl[:, a * H:(a + 1) * H] + gae[:, a * AE:a * AE + H],
                jnp.bfloat16(0.0),
            )
            for a in range(A)
        ],
        axis=-1,
    )
    logits = jnp.dot(h, w2b_ref[...], preferred_element_type=jnp.float32)
    logits = jnp.where(m_ref[...] > 0.0, logits, VERY_NEG)
    mx = jnp.max(logits, axis=-1, keepdims=True)
    e = jnp.exp(logits - mx)
    attn = e / jnp.sum(e, axis=-1, keepdims=True)
    out_ref[...] = functools.reduce(
        lambda x, y: x + y,
        [
            attn[:, a:a + 1]
            * gae[:, a * AE + H:(a + 1) * AE].astype(jnp.float32)
            for a in range(A)
        ],
    )


def _attention(gl, gae, masks, w2_blk):
    nc = gl.shape[0]
    return pl.pallas_call(
        _attn_kernel,
        grid=(nc // TN,),
        in_specs=[
            pl.BlockSpec((TN, A * H), lambda i: (i, 0)),
            pl.BlockSpec((TN, A * AE), lambda i: (i, 0)),
            pl.BlockSpec((TN, A), lambda i: (i, 0)),
            pl.BlockSpec((A * H, A), lambda i: (0, 0)),
        ],
        out_specs=pl.BlockSpec((TN, OUT), lambda i: (i, 0)),
        out_shape=jax.ShapeDtypeStruct((nc, OUT), jnp.float32),
        compiler_params=pltpu.CompilerParams(
            dimension_semantics=("parallel",),
            vmem_limit_bytes=100 * 1024 * 1024,
        ),
    )(gl, gae, masks, w2_blk)


P4 = OUT // 128


def _pool_kernel(idx_ref, wgt_ref, cm_ref, c4_ref, bc_ref, out_ref):
    def row(r, _):
        acc = jnp.zeros((P4, 128), jnp.float32)
        cnt = jnp.float32(0.0)
        for m in range(48):
            i4 = pl.multiple_of(idx_ref[r, m], P4)
            acc = acc + wgt_ref[r, m] * c4_ref[pl.ds(i4, P4), :]
            cnt = cnt + cm_ref[r, m]
        scale = 1.0 / jnp.maximum(jnp.full((P4, 128), cnt), 1.0)
        out_ref[pl.ds(r * P4, P4), :] = acc * scale + bc_ref[...]
        return ()

    jax.lax.fori_loop(0, TB, row, ())


def _pool(idx4, wgt, cmf, c4, bc4):
    bv = idx4.shape[0]
    return pl.pallas_call(
        _pool_kernel,
        grid=(bv // TB,),
        in_specs=[
            pl.BlockSpec((TB, 48), lambda i: (i, 0),
                         memory_space=pltpu.SMEM),
            pl.BlockSpec((TB, 48), lambda i: (i, 0),
                         memory_space=pltpu.SMEM),
            pl.BlockSpec((TB, 48), lambda i: (i, 0),
                         memory_space=pltpu.SMEM),
            pl.BlockSpec((N_CODES * P4, 128), lambda i: (0, 0)),
            pl.BlockSpec((P4, 128), lambda i: (0, 0)),
        ],
        out_specs=pl.BlockSpec((TB * P4, 128), lambda i: (i, 0)),
        out_shape=jax.ShapeDtypeStruct((bv * P4, 128), jnp.float32),
        compiler_params=pltpu.CompilerParams(
            dimension_semantics=("parallel",),
            vmem_limit_bytes=100 * 1024 * 1024,
        ),
    )(idx4, wgt, cmf, c4, bc4)


def kernel(embed_table, w1, b1, w2, b2, wc, bc, masks, code_mask,
           leaves_list, ancestors_list, input_ids):
    del b2
    f32 = jnp.float32
    w_cat = jnp.concatenate([w1[:H, :], w1[H:, :], wc],
                            axis=1).astype(jnp.bfloat16)
    b_cat = jnp.concatenate(
        [jnp.zeros((H,), f32), b1, jnp.zeros((OUT,), f32)]).reshape(1, H + AE)
    ql, qae = _project(embed_table, w_cat, b_cat)

    w2b = jnp.zeros((A, H, A), f32).at[jnp.arange(A), :, jnp.arange(A)].set(
        w2[:, 0][None, :]).reshape(A * H, A).astype(jnp.bfloat16)

    leaves = leaves_list.astype(jnp.int32)
    anc = ancestors_list.astype(jnp.int32)
    nc = N_CODES // NCHUNK
    chunks = []
    for c in range(NCHUNK):
        sl = slice(c * nc, (c + 1) * nc)
        gl = ql[leaves[sl]].reshape(nc, A * H)
        gae = qae[anc[sl]].reshape(nc, A * AE)
        chunks.append(_attention(gl, gae, masks[sl], w2b))
    ctab = jnp.concatenate(chunks, axis=0)

    ids = input_ids.reshape(-1, 48).astype(jnp.int32)
    idx4 = jnp.where(ids == 0, 0, (ids - 1) * P4)
    cmf = code_mask.reshape(-1, 48).astype(f32)
    wgt = cmf * (ids != 0).astype(f32)
    out = _pool(idx4, wgt, cmf, ctab.reshape(N_CODES * P4, 128),
                bc.reshape(P4, 128))
    B, V, _ = input_ids.shape
    return out.reshape(B, V, OUT)


# speedup vs baseline: 1.8532x; 1.3997x over previous
import jax
import jax.numpy as jnp
from jax.experimental import pallas as pl
from jax.experimental.pallas import tpu as pltpu

H = 768
A = 8
AE = H + 512
N_CODES = 10000
OUT = 512
VERY_NEG = -1e30

TM = 1000
TN = 200
NCHUNK = 5
TB = 64


def _proj_kernel(e_ref, w_ref, b_ref, o_ref):
    eb = e_ref[...].astype(jnp.bfloat16)
    p = jnp.dot(eb, w_ref[...], preferred_element_type=jnp.float32)
    o_ref[...] = (p + b_ref[...]).astype(jnp.bfloat16)


def _project(embed_table, w, b):
    n, d = embed_table.shape[0], w.shape[1]
    return pl.pallas_call(
        _proj_kernel,
        grid=(n // TM,),
        in_specs=[
            pl.BlockSpec((TM, H), lambda i: (i, 0)),
            pl.BlockSpec((H, d), lambda i: (0, 0)),
            pl.BlockSpec((1, d), lambda i: (0, 0)),
        ],
        out_specs=pl.BlockSpec((TM, d), lambda i: (i, 0)),
        out_shape=jax.ShapeDtypeStruct((n, d), jnp.bfloat16),
        compiler_params=pltpu.CompilerParams(
            dimension_semantics=("parallel",),
            vmem_limit_bytes=100 * 1024 * 1024,
        ),
    )(embed_table, w, b)


def _attn_kernel(gl_ref, gae_ref, m3_ref, w2t_ref, out_ref):
    gl = gl_ref[...]
    gae = gae_ref[...]
    h = jnp.maximum(gl + gae[:, :H], jnp.bfloat16(0.0))
    lg = jnp.dot(h, w2t_ref[...], preferred_element_type=jnp.float32)
    lg3 = lg.reshape(TN, A, A)
    lg3 = jnp.where(m3_ref[...] > 0.0, lg3, VERY_NEG)
    mx = jnp.max(lg3, axis=1, keepdims=True)
    e3 = jnp.exp(lg3 - mx)
    attn3 = e3 / jnp.sum(e3, axis=1, keepdims=True)
    ew3 = gae[:, H:].reshape(TN, A, OUT).astype(jnp.float32)
    out_ref[...] = jnp.sum(ew3 * attn3[:, :, :1], axis=1)


def _attention(gl, gae, mask3, w2t, c0, nc):
    return pl.pallas_call(
        _attn_kernel,
        grid=(nc // TN,),
        in_specs=[
            pl.BlockSpec((A * TN, H), lambda i: (c0 + i, 0)),
            pl.BlockSpec((A * TN, AE), lambda i: (i, 0)),
            pl.BlockSpec((TN, A, A), lambda i: (c0 + i, 0, 0)),
            pl.BlockSpec((H, A), lambda i: (0, 0)),
        ],
        out_specs=pl.BlockSpec((TN, OUT), lambda i: (i, 0)),
        out_shape=jax.ShapeDtypeStruct((nc, OUT), jnp.float32),
        compiler_params=pltpu.CompilerParams(
            dimension_semantics=("parallel",),
            vmem_limit_bytes=100 * 1024 * 1024,
        ),
    )(gl, gae, mask3, w2t)


P4 = OUT // 128


def _pool_kernel(idx_ref, wgt_ref, cm_ref, c4_ref, bc_ref, out_ref):
    def row(r, _):
        acc = jnp.zeros((P4, 128), jnp.float32)
        cnt = jnp.float32(0.0)
        for m in range(48):
            i4 = pl.multiple_of(idx_ref[r, m], P4)
            acc = acc + wgt_ref[r, m] * c4_ref[pl.ds(i4, P4), :]
            cnt = cnt + cm_ref[r, m]
        scale = 1.0 / jnp.maximum(jnp.full((P4, 128), cnt), 1.0)
        out_ref[pl.ds(r * P4, P4), :] = acc * scale + bc_ref[...]
        return ()

    jax.lax.fori_loop(0, TB, row, ())


def _pool(idx4, wgt, cmf, c4, bc4):
    bv = idx4.shape[0]
    return pl.pallas_call(
        _pool_kernel,
        grid=(bv // TB,),
        in_specs=[
            pl.BlockSpec((TB, 48), lambda i: (i, 0),
                         memory_space=pltpu.SMEM),
            pl.BlockSpec((TB, 48), lambda i: (i, 0),
                         memory_space=pltpu.SMEM),
            pl.BlockSpec((TB, 48), lambda i: (i, 0),
                         memory_space=pltpu.SMEM),
            pl.BlockSpec((N_CODES * P4, 128), lambda i: (0, 0)),
            pl.BlockSpec((P4, 128), lambda i: (0, 0)),
        ],
        out_specs=pl.BlockSpec((TB * P4, 128), lambda i: (i, 0)),
        out_shape=jax.ShapeDtypeStruct((bv * P4, 128), jnp.float32),
        compiler_params=pltpu.CompilerParams(
            dimension_semantics=("parallel",),
            vmem_limit_bytes=100 * 1024 * 1024,
        ),
    )(idx4, wgt, cmf, c4, bc4)


def kernel(embed_table, w1, b1, w2, b2, wc, bc, masks, code_mask,
           leaves_list, ancestors_list, input_ids):
    del b2
    f32 = jnp.float32
    w_l = w1[:H, :].astype(jnp.bfloat16)
    w_ae = jnp.concatenate([w1[H:, :], wc], axis=1).astype(jnp.bfloat16)
    b_l = jnp.zeros((1, H), f32)
    b_ae = jnp.concatenate([b1, jnp.zeros((OUT,), f32)]).reshape(1, AE)
    ql = _project(embed_table, w_l, b_l)
    qae = _project(embed_table, w_ae, b_ae)

    gl = ql[leaves_list.reshape(-1).astype(jnp.int32)]
    anc_flat = ancestors_list.reshape(-1).astype(jnp.int32)
    mask3 = jnp.broadcast_to(masks[:, :, None], (N_CODES, A, A))
    w2t = jnp.broadcast_to(w2, (H, A)).astype(jnp.bfloat16)
    nc = N_CODES // NCHUNK
    chunks = []
    for c in range(NCHUNK):
        gae = qae[anc_flat[c * nc * A:(c + 1) * nc * A]]
        chunks.append(
            _attention(gl, gae, mask3, w2t, c * (nc // TN), nc))
    ctab = jnp.concatenate(chunks, axis=0)

    ids = input_ids.reshape(-1, 48).astype(jnp.int32)
    idx4 = jnp.where(ids == 0, 0, (ids - 1) * P4)
    cmf = code_mask.reshape(-1, 48).astype(f32)
    wgt = cmf * (ids != 0).astype(f32)
    out = _pool(idx4, wgt, cmf, ctab.reshape(N_CODES * P4, 128),
                bc.reshape(P4, 128))
    B, V, _ = input_ids.shape
    return out.reshape(B, V, OUT)
